# baseline (speedup 1.0000x reference)
import functools

import jax
import jax.numpy as jnp
from jax.experimental import pallas as pl
from jax.experimental.pallas import tpu as pltpu

_GAMMA = 1.0
_LAM = 1.0
_N_ITER = 4
_LR = 0.02
_EPS = 1e-3
_BISECT_ITERS = 30
_JT = 512


def _fusedmax_body(xc_ref, a_hbm, o_ref, a_vmem, sems):
    b = pl.program_id(0)
    nb = pl.num_programs(0)
    slot = jax.lax.rem(b, 2)
    nxt = jax.lax.rem(b + 1, 2)

    @pl.when(b == 0)
    def _():
        pltpu.make_async_copy(a_hbm.at[0], a_vmem.at[0], sems.at[0]).start()

    @pl.when(b + 1 < nb)
    def _():
        pltpu.make_async_copy(a_hbm.at[b + 1], a_vmem.at[nxt],
                              sems.at[nxt]).start()

    pltpu.make_async_copy(a_hbm.at[b], a_vmem.at[slot], sems.at[slot]).wait()

    x_col = xc_ref[0]
    y = x_col
    M = x_col.shape[0]
    nt = M // _JT
    r_io = jax.lax.broadcasted_iota(jnp.int32, (_JT, _JT), 0)
    c_io = jax.lax.broadcasted_iota(jnp.int32, (_JT, _JT), 1)
    triu = r_io < c_io

    def sl(t):
        return slice(t * _JT, (t + 1) * _JT)

    for _ in range(_N_ITER):
        u = y * (1.0 / _EPS)
        ut = jnp.transpose(u)
        rs = [None] * nt
        for ti in range(nt):
            for tj in range(nt):
                a_t = a_vmem[slot, sl(ti), sl(tj)]
                d = u[sl(ti)] - ut[:, sl(tj)]
                p = a_t * jnp.tanh(d)
                prs = jnp.sum(p, axis=1, keepdims=True)
                rs[ti] = prs if rs[ti] is None else rs[ti] + prs
        pen = jnp.concatenate(rs, axis=0)
        y = y - _LR * ((y - x_col) + _LAM * pen)

    z = jnp.transpose(y) * (1.0 / _GAMMA)
    zmax = jnp.max(z)
    lo = zmax - 1.0
    hi = zmax

    def bis(_, carry):
        lo, hi = carry
        mid = 0.5 * (lo + hi)
        f = jnp.sum(jnp.maximum(z - mid, 0.0))
        gt = f > 1.0
        return jnp.where(gt, mid, lo), jnp.where(gt, hi, mid)

    lo, hi = jax.lax.fori_loop(0, _BISECT_ITERS, bis, (lo, hi))
    tau = 0.5 * (lo + hi)
    o_ref[0] = jnp.maximum(jnp.transpose(y) - tau, 0.0)


@jax.jit
def kernel(x, A):
    B, M = x.shape
    xt3 = x.reshape(B, M, 1)
    out = pl.pallas_call(
        _fusedmax_body,
        grid=(B,),
        in_specs=[
            pl.BlockSpec((1, M, 1), lambda b: (b, 0, 0)),
            pl.BlockSpec(memory_space=pltpu.MemorySpace.HBM),
        ],
        out_specs=pl.BlockSpec((1, 1, M), lambda b: (b, 0, 0)),
        out_shape=jax.ShapeDtypeStruct((B, 1, M), x.dtype),
        scratch_shapes=[
            pltpu.VMEM((2, M, M), jnp.float32),
            pltpu.SemaphoreType.DMA((2,)),
        ],
    )(xt3, A)
    return out.reshape(B, M)


# speedup vs baseline: 1.1506x; 1.1506x over previous
import functools

import jax
import jax.numpy as jnp
from jax.experimental import pallas as pl
from jax.experimental.pallas import tpu as pltpu

_GAMMA = 1.0
_LAM = 1.0
_N_ITER = 4
_LR = 0.02
_EPS = 1e-3
_BISECT_ITERS = 30
_JT = 1024


def _fusedmax_body(xc_ref, a_ref, o_ref):
    x_col = xc_ref[0]
    y = x_col
    M = x_col.shape[0]
    nt = M // _JT
    r_io = jax.lax.broadcasted_iota(jnp.int32, (_JT, _JT), 0)
    c_io = jax.lax.broadcasted_iota(jnp.int32, (_JT, _JT), 1)
    triu = r_io < c_io

    def sl(t):
        return slice(t * _JT, (t + 1) * _JT)

    for _ in range(_N_ITER):
        u = y * (1.0 / _EPS)
        ut = jnp.transpose(u)
        rs = [None] * nt
        cs = [None] * nt
        for ti in range(nt):
            for tj in range(ti, nt):
                a_t = a_ref[0, sl(ti), sl(tj)]
                d = u[sl(ti)] - ut[:, sl(tj)]
                p = a_t * jnp.tanh(d)
                if ti == tj:
                    p = jnp.where(triu, p, 0.0)
                prs = jnp.sum(p, axis=1, keepdims=True)
                pcs = jnp.sum(p, axis=0, keepdims=True)
                rs[ti] = prs if rs[ti] is None else rs[ti] + prs
                cs[tj] = pcs if cs[tj] is None else cs[tj] + pcs
        rsv = jnp.concatenate(rs, axis=0)
        csv = jnp.concatenate(cs, axis=1)
        pen = rsv - jnp.transpose(csv)
        y = y - _LR * ((y - x_col) + _LAM * pen)

    z = jnp.transpose(y) * (1.0 / _GAMMA)
    zmax = jnp.max(z)
    lo = zmax - 1.0
    hi = zmax

    def bis(_, carry):
        lo, hi = carry
        mid = 0.5 * (lo + hi)
        f = jnp.sum(jnp.maximum(z - mid, 0.0))
        gt = f > 1.0
        return jnp.where(gt, mid, lo), jnp.where(gt, hi, mid)

    lo, hi = jax.lax.fori_loop(0, _BISECT_ITERS, bis, (lo, hi))
    tau = 0.5 * (lo + hi)
    o_ref[0] = jnp.maximum(jnp.transpose(y) - tau, 0.0)


@jax.jit
def kernel(x, A):
    B, M = x.shape
    xt3 = x.reshape(B, M, 1)
    grid_spec = pl.GridSpec(
        grid=(B,),
        in_specs=[
            pl.BlockSpec((1, M, 1), lambda b: (b, 0, 0)),
            pl.BlockSpec((1, M, M), lambda b: (b, 0, 0)),
        ],
        out_specs=pl.BlockSpec((1, 1, M), lambda b: (b, 0, 0)),
    )
    out = pl.pallas_call(
        _fusedmax_body,
        grid_spec=grid_spec,
        out_shape=jax.ShapeDtypeStruct((B, 1, M), x.dtype),
    )(xt3, A)
    return out.reshape(B, M)
